# baseline (speedup 1.0000x reference)
import functools

import jax
import jax.numpy as jnp
from jax import lax
from jax.experimental import pallas as pl
from jax.experimental.pallas import tpu as pltpu
from jax.experimental.pallas import tpu_sc as plsc

H = 4
C = 64
HC = H * C
ED = 16
AW = 128
G = 128
NTILES = 32


def _p1_body(x_ref, wl_ref, bl_ref, wr_ref, br_ref, xl_ref, xr_ref):
    xb = x_ref[...]
    xl_ref[...] = jnp.dot(xb, wl_ref[...], preferred_element_type=jnp.float32) + bl_ref[...]
    xr_ref[...] = jnp.dot(xb, wr_ref[...], preferred_element_type=jnp.float32) + br_ref[...]


def _make_gather(e_pad):
    n_chunks = e_pad // G
    mesh = plsc.VectorSubcoreMesh(core_axis_name="c", subcore_axis_name="s")

    c0_chunks = (n_chunks * 7 // 20) // 16 * 16

    @functools.partial(
        pl.kernel,
        mesh=mesh,
        out_type=jax.ShapeDtypeStruct((e_pad, HC), jnp.float32),
    )
    def gather_k(tbl_hbm, idx_hbm, o_hbm):
        cid = lax.axis_index("c")

        def pipe(n_ch, ch0):
            def body(idx_vmem, o_vmem):
                pltpu.sync_copy(tbl_hbm.at[idx_vmem.at[0]], o_vmem)

            pltpu.emit_pipeline(
                body,
                grid=(n_ch,),
                in_specs=[pl.BlockSpec((1, G), lambda i: (0, i + ch0))],
                out_specs=[pl.BlockSpec((G, HC), lambda i: (i + ch0, 0))],
                core_axis_name="s",
                dimension_semantics=(pltpu.PARALLEL,),
            )(idx_hbm, o_hbm)

        @pl.when(cid == 0)
        def _():
            pipe(c0_chunks, 0)

        @pl.when(cid == 1)
        def _():
            pipe(n_chunks - c0_chunks, c0_chunks)

    return gather_k


def _p3_body(xl_ref, xr_ref, ea_ref, we_ref, attbd_ref, w1_ref, b1_ref,
             w2_ref, b2_ref, aux_ref, m_ref, m_acc):
    i = pl.program_id(0)
    ea = ea_ref[...]
    ee = jnp.dot(ea.astype(jnp.bfloat16), we_ref[...],
                 preferred_element_type=jnp.float32)
    v = xl_ref[...] + xr_ref[...] + ee
    v = jnp.where(v >= 0, v, 0.2 * v)
    alpha8 = jnp.dot(v.astype(jnp.bfloat16), attbd_ref[...],
                     preferred_element_type=jnp.float32)
    alpha = alpha8[:, 0:4]
    g1 = jnp.dot(ea.astype(jnp.bfloat16), w1_ref[...],
                 preferred_element_type=jnp.float32) + b1_ref[...]
    g1 = g1 * jax.nn.sigmoid(g1)
    g2 = jnp.sum(g1 * w2_ref[...], axis=1, keepdims=True) + b2_ref[...]
    gate = jax.nn.sigmoid(g2)
    one = jnp.ones_like(gate)
    zero = jnp.zeros_like(gate)
    aux_ref[...] = jnp.concatenate([alpha, gate, one, zero, zero], axis=1)
    blkmax = jnp.max(alpha)

    @pl.when(i == 0)
    def _():
        m_acc[0, 0] = blkmax

    @pl.when(i > 0)
    def _():
        m_acc[0, 0] = jnp.maximum(m_acc[0, 0], blkmax)

    m_ref[...] = jnp.full((1, 1), m_acc[0, 0], jnp.float32)


def _p4_body(e_real, be, aux_ref, xl_ref, m_ref, msg0_ref, msg1_ref, den_ref):
    i = pl.program_id(0)
    mglob = m_ref[...]
    aux = aux_ref[...]
    xl = xl_ref[...]
    rows = i * be + lax.broadcasted_iota(jnp.int32, (be, 1), 0)
    valid = (rows < e_real).astype(jnp.float32)
    a = jnp.exp(aux[:, 0:4] - mglob) * valid
    gate = aux[:, 4:5] * valid
    one = aux[:, 5:6] * valid
    msg0_ref[...] = jnp.concatenate(
        [xl[:, 0:64] * a[:, 0:1], xl[:, 64:128] * a[:, 1:2]], axis=1)
    msg1_ref[...] = jnp.concatenate(
        [xl[:, 128:192] * a[:, 2:3], xl[:, 192:256] * a[:, 3:4]], axis=1)
    den_ref[...] = jnp.concatenate(
        [a, gate, one, jnp.zeros((be, AW - 6), jnp.float32)], axis=1)


def _scatter_pipe(table_sh, dst2_hbm, msg_hbm, n_chunks, chunk0):
    def body(idx_vmem, msg_vmem):
        pltpu.sync_copy(msg_vmem, table_sh.at[idx_vmem.at[0]], add=True)

    pltpu.emit_pipeline(
        body,
        grid=(n_chunks,),
        in_specs=[pl.BlockSpec((1, G), lambda i: (0, i + chunk0)),
                  pl.BlockSpec((G, AW), lambda i: (i + chunk0, 0))],
        core_axis_name="s",
        dimension_semantics=(pltpu.PARALLEL,),
    )(dst2_hbm, msg_hbm)


def _make_scatter(n_nodes, e_pad):
    n_chunks = e_pad // G
    mesh = plsc.VectorSubcoreMesh(core_axis_name="c", subcore_axis_name="s")

    @functools.partial(
        pl.kernel,
        mesh=mesh,
        out_type=[jax.ShapeDtypeStruct((n_nodes, AW), jnp.float32),
                  jax.ShapeDtypeStruct((n_nodes, AW), jnp.float32)],
        scratch_types=[pltpu.VMEM_SHARED((n_nodes, AW), jnp.float32)],
    )
    def scatter_k(msg0_hbm, msg1_hbm, dst2_hbm, zeros_hbm, o0_hbm, o1_hbm,
                  table_sh):
        cid = lax.axis_index("c")
        sid = lax.axis_index("s")

        @pl.when(sid == 0)
        def _():
            pltpu.sync_copy(zeros_hbm, table_sh)

        plsc.subcore_barrier()

        @pl.when(cid == 0)
        def _():
            _scatter_pipe(table_sh, dst2_hbm, msg0_hbm, n_chunks, 0)

        @pl.when(cid == 1)
        def _():
            _scatter_pipe(table_sh, dst2_hbm, msg1_hbm, n_chunks, 0)

        plsc.subcore_barrier()

        @pl.when((sid == 0) & (cid == 0))
        def _():
            pltpu.sync_copy(table_sh, o0_hbm)

        @pl.when((sid == 0) & (cid == 1))
        def _():
            pltpu.sync_copy(table_sh, o1_hbm)

    return scatter_k


def _make_den_scatter(n_nodes, e_pad):
    half_chunks = e_pad // (2 * G)
    mesh = plsc.VectorSubcoreMesh(core_axis_name="c", subcore_axis_name="s")

    @functools.partial(
        pl.kernel,
        mesh=mesh,
        out_type=[jax.ShapeDtypeStruct((n_nodes, AW), jnp.float32),
                  jax.ShapeDtypeStruct((n_nodes, AW), jnp.float32)],
        scratch_types=[pltpu.VMEM_SHARED((n_nodes, AW), jnp.float32)],
    )
    def den_k(den_hbm, dst2_hbm, zeros_hbm, o0_hbm, o1_hbm, table_sh):
        cid = lax.axis_index("c")
        sid = lax.axis_index("s")

        @pl.when(sid == 0)
        def _():
            pltpu.sync_copy(zeros_hbm, table_sh)

        plsc.subcore_barrier()

        @pl.when(cid == 0)
        def _():
            _scatter_pipe(table_sh, dst2_hbm, den_hbm, half_chunks, 0)

        @pl.when(cid == 1)
        def _():
            _scatter_pipe(table_sh, dst2_hbm, den_hbm, half_chunks, half_chunks)

        plsc.subcore_barrier()

        @pl.when((sid == 0) & (cid == 0))
        def _():
            pltpu.sync_copy(table_sh, o0_hbm)

        @pl.when((sid == 0) & (cid == 1))
        def _():
            pltpu.sync_copy(table_sh, o1_hbm)

    return den_k


def _p6_body(acc0_ref, acc1_ref, den0_ref, den1_ref, x_ref, bias_ref,
             lnw_ref, lnb_ref, o_ref):
    a0 = acc0_ref[...]
    a1 = acc1_ref[...]
    bn = a0.shape[0]
    d8 = den0_ref[:, 0:8] + den1_ref[:, 0:8]
    num = jnp.concatenate([a0, a1], axis=1)
    den = jnp.concatenate(
        [jnp.broadcast_to(d8[:, h:h + 1], (bn, C)) for h in range(H)], axis=1)
    out = num / (den + 1e-30) + bias_ref[...]
    mean_gate = d8[:, 4:5] / jnp.maximum(d8[:, 5:6], 1.0)
    out = out * mean_gate
    mu = jnp.mean(out, axis=-1, keepdims=True)
    var = jnp.mean((out - mu) ** 2, axis=-1, keepdims=True)
    out = (out - mu) * lax.rsqrt(var + 1e-5) * lnw_ref[...] + lnb_ref[...]
    out = out * jax.nn.sigmoid(out)
    o_ref[...] = out + x_ref[...]


def kernel(x, edge_index, edge_attr, W_l, b_l, W_r, b_r, W_e, att, bias,
           eg_W1, eg_b1, eg_W2, eg_b2, ln_w, ln_b):
    n = x.shape[0]
    e = edge_attr.shape[0]
    e_pad = ((e + NTILES * G - 1) // (NTILES * G)) * (NTILES * G)
    pad = e_pad - e
    src_p = jnp.concatenate([edge_index[0].astype(jnp.int32),
                             jnp.zeros((pad,), jnp.int32)])
    dst_p = jnp.concatenate([edge_index[1].astype(jnp.int32),
                             jnp.zeros((pad,), jnp.int32)])
    ea_p = jnp.concatenate([edge_attr, jnp.zeros((pad, ED), edge_attr.dtype)])

    BN = 1000
    xl, xr = pl.pallas_call(
        _p1_body,
        out_shape=[jax.ShapeDtypeStruct((n, HC), jnp.float32)] * 2,
        grid=(n // BN,),
        in_specs=[
            pl.BlockSpec((BN, HC), lambda i: (i, 0)),
            pl.BlockSpec((HC, HC), lambda i: (0, 0)),
            pl.BlockSpec((1, HC), lambda i: (0, 0)),
            pl.BlockSpec((HC, HC), lambda i: (0, 0)),
            pl.BlockSpec((1, HC), lambda i: (0, 0)),
        ],
        out_specs=[pl.BlockSpec((BN, HC), lambda i: (i, 0))] * 2,
    )(x, W_l, b_l.reshape(1, HC), W_r, b_r.reshape(1, HC))

    gather = _make_gather(e_pad)
    xl_src = gather(xl, src_p.reshape(1, e_pad))
    xr_dst = gather(xr, dst_p.reshape(1, e_pad))

    BE = 2048
    n_eblk = e_pad // BE
    idx256 = jnp.arange(HC)
    attbd = jnp.where(idx256[:, None] // C == jnp.arange(8)[None, :],
                      att.reshape(HC)[:, None], 0.0).astype(jnp.bfloat16)
    aux, mglob = pl.pallas_call(
        _p3_body,
        out_shape=[jax.ShapeDtypeStruct((e_pad, 8), jnp.float32),
                   jax.ShapeDtypeStruct((1, 1), jnp.float32)],
        grid=(n_eblk,),
        in_specs=[
            pl.BlockSpec((BE, HC), lambda i: (i, 0)),
            pl.BlockSpec((BE, HC), lambda i: (i, 0)),
            pl.BlockSpec((BE, ED), lambda i: (i, 0)),
            pl.BlockSpec((ED, HC), lambda i: (0, 0)),
            pl.BlockSpec((HC, 8), lambda i: (0, 0)),
            pl.BlockSpec((ED, 2 * ED), lambda i: (0, 0)),
            pl.BlockSpec((1, 2 * ED), lambda i: (0, 0)),
            pl.BlockSpec((1, 2 * ED), lambda i: (0, 0)),
            pl.BlockSpec((1, 1), lambda i: (0, 0)),
        ],
        out_specs=[pl.BlockSpec((BE, 8), lambda i: (i, 0)),
                   pl.BlockSpec((1, 1), lambda i: (0, 0))],
        scratch_shapes=[pltpu.SMEM((1, 1), jnp.float32)],
    )(xl_src, xr_dst, ea_p, W_e.astype(jnp.bfloat16), attbd,
      eg_W1.astype(jnp.bfloat16), eg_b1.reshape(1, 2 * ED),
      eg_W2.reshape(1, 2 * ED), eg_b2.reshape(1, 1))

    msg0, msg1, denrows = pl.pallas_call(
        functools.partial(_p4_body, e, BE),
        out_shape=[jax.ShapeDtypeStruct((e_pad, AW), jnp.float32)] * 3,
        grid=(n_eblk,),
        in_specs=[
            pl.BlockSpec((BE, 8), lambda i: (i, 0)),
            pl.BlockSpec((BE, HC), lambda i: (i, 0)),
            pl.BlockSpec((1, 1), lambda i: (0, 0)),
        ],
        out_specs=[pl.BlockSpec((BE, AW), lambda i: (i, 0))] * 3,
    )(aux, xl_src, mglob)

    zeros_tbl = jnp.zeros((n, AW), jnp.float32)
    dst2 = dst_p.reshape(1, e_pad)
    acc0, acc1 = _make_scatter(n, e_pad)(msg0, msg1, dst2, zeros_tbl)
    den0, den1 = _make_den_scatter(n, e_pad)(denrows, dst2, zeros_tbl)

    out = pl.pallas_call(
        _p6_body,
        out_shape=jax.ShapeDtypeStruct((n, HC), jnp.float32),
        grid=(n // BN,),
        in_specs=[
            pl.BlockSpec((BN, AW), lambda i: (i, 0)),
            pl.BlockSpec((BN, AW), lambda i: (i, 0)),
            pl.BlockSpec((BN, AW), lambda i: (i, 0)),
            pl.BlockSpec((BN, AW), lambda i: (i, 0)),
            pl.BlockSpec((BN, HC), lambda i: (i, 0)),
            pl.BlockSpec((1, HC), lambda i: (0, 0)),
            pl.BlockSpec((1, HC), lambda i: (0, 0)),
            pl.BlockSpec((1, HC), lambda i: (0, 0)),
        ],
        out_specs=pl.BlockSpec((BN, HC), lambda i: (i, 0)),
    )(acc0, acc1, den0, den1, x, bias.reshape(1, HC), ln_w.reshape(1, HC),
      ln_b.reshape(1, HC))
    return out


# speedup vs baseline: 1.0165x; 1.0165x over previous
import functools

import jax
import jax.numpy as jnp
from jax import lax
from jax.experimental import pallas as pl
from jax.experimental.pallas import tpu as pltpu
from jax.experimental.pallas import tpu_sc as plsc

H = 4
C = 64
HC = H * C
ED = 16
AW = 128
G = 128
NTILES = 32


def _p1_body(x_ref, wl_ref, bl_ref, wr_ref, br_ref, xl_ref, xr_ref):
    xb = x_ref[...]
    xl_ref[...] = jnp.dot(xb, wl_ref[...], preferred_element_type=jnp.float32) + bl_ref[...]
    xr_ref[...] = jnp.dot(xb, wr_ref[...], preferred_element_type=jnp.float32) + br_ref[...]


def _make_gather(e_pad):
    n_chunks = e_pad // G
    mesh = plsc.VectorSubcoreMesh(core_axis_name="c", subcore_axis_name="s")

    c0_chunks = (n_chunks * 13 // 20) // 16 * 16

    @functools.partial(
        pl.kernel,
        mesh=mesh,
        out_type=jax.ShapeDtypeStruct((e_pad, HC), jnp.float32),
    )
    def gather_k(tbl_hbm, idx_hbm, o_hbm):
        cid = lax.axis_index("c")

        def pipe(n_ch, ch0):
            def body(idx_vmem, o_vmem):
                pltpu.sync_copy(tbl_hbm.at[idx_vmem.at[0]], o_vmem)

            pltpu.emit_pipeline(
                body,
                grid=(n_ch,),
                in_specs=[pl.BlockSpec((1, G), lambda i: (0, i + ch0))],
                out_specs=[pl.BlockSpec((G, HC), lambda i: (i + ch0, 0))],
                core_axis_name="s",
                dimension_semantics=(pltpu.PARALLEL,),
            )(idx_hbm, o_hbm)

        @pl.when(cid == 0)
        def _():
            pipe(c0_chunks, 0)

        @pl.when(cid == 1)
        def _():
            pipe(n_chunks - c0_chunks, c0_chunks)

    return gather_k


def _p3_body(xl_ref, xr_ref, ea_ref, we_ref, attbd_ref, w1_ref, b1_ref,
             w2_ref, b2_ref, aux_ref, m_ref, m_acc):
    i = pl.program_id(0)
    ea = ea_ref[...]
    ee = jnp.dot(ea.astype(jnp.bfloat16), we_ref[...],
                 preferred_element_type=jnp.float32)
    v = xl_ref[...] + xr_ref[...] + ee
    v = jnp.where(v >= 0, v, 0.2 * v)
    alpha8 = jnp.dot(v.astype(jnp.bfloat16), attbd_ref[...],
                     preferred_element_type=jnp.float32)
    alpha = alpha8[:, 0:4]
    g1 = jnp.dot(ea.astype(jnp.bfloat16), w1_ref[...],
                 preferred_element_type=jnp.float32) + b1_ref[...]
    g1 = g1 * jax.nn.sigmoid(g1)
    g2 = jnp.sum(g1 * w2_ref[...], axis=1, keepdims=True) + b2_ref[...]
    gate = jax.nn.sigmoid(g2)
    one = jnp.ones_like(gate)
    zero = jnp.zeros_like(gate)
    aux_ref[...] = jnp.concatenate([alpha, gate, one, zero, zero], axis=1)
    blkmax = jnp.max(alpha)

    @pl.when(i == 0)
    def _():
        m_acc[0, 0] = blkmax

    @pl.when(i > 0)
    def _():
        m_acc[0, 0] = jnp.maximum(m_acc[0, 0], blkmax)

    m_ref[...] = jnp.full((1, 1), m_acc[0, 0], jnp.float32)


def _p4_body(e_real, be, aux_ref, xl_ref, m_ref, msg0_ref, msg1_ref, den_ref):
    i = pl.program_id(0)
    mglob = m_ref[...]
    aux = aux_ref[...]
    xl = xl_ref[...]
    rows = i * be + lax.broadcasted_iota(jnp.int32, (be, 1), 0)
    valid = (rows < e_real).astype(jnp.float32)
    a = jnp.exp(aux[:, 0:4] - mglob) * valid
    gate = aux[:, 4:5] * valid
    one = aux[:, 5:6] * valid
    msg0_ref[...] = jnp.concatenate(
        [xl[:, 0:64] * a[:, 0:1], xl[:, 64:128] * a[:, 1:2]], axis=1)
    msg1_ref[...] = jnp.concatenate(
        [xl[:, 128:192] * a[:, 2:3], xl[:, 192:256] * a[:, 3:4]], axis=1)
    den_ref[...] = jnp.concatenate(
        [a, gate, one, jnp.zeros((be, AW - 6), jnp.float32)], axis=1)


def _scatter_pipe(table_sh, dst2_hbm, msg_hbm, n_chunks, chunk0):
    def body(idx_vmem, msg_vmem):
        pltpu.sync_copy(msg_vmem, table_sh.at[idx_vmem.at[0]], add=True)

    pltpu.emit_pipeline(
        body,
        grid=(n_chunks,),
        in_specs=[pl.BlockSpec((1, G), lambda i: (0, i + chunk0)),
                  pl.BlockSpec((G, AW), lambda i: (i + chunk0, 0))],
        core_axis_name="s",
        dimension_semantics=(pltpu.PARALLEL,),
    )(dst2_hbm, msg_hbm)


def _make_scatter(n_nodes, e_pad):
    n_chunks = e_pad // G
    mesh = plsc.VectorSubcoreMesh(core_axis_name="c", subcore_axis_name="s")

    @functools.partial(
        pl.kernel,
        mesh=mesh,
        out_type=[jax.ShapeDtypeStruct((n_nodes, AW), jnp.float32),
                  jax.ShapeDtypeStruct((n_nodes, AW), jnp.float32)],
        scratch_types=[pltpu.VMEM_SHARED((n_nodes, AW), jnp.float32)],
    )
    def scatter_k(msg0_hbm, msg1_hbm, dst2_hbm, zeros_hbm, o0_hbm, o1_hbm,
                  table_sh):
        cid = lax.axis_index("c")
        sid = lax.axis_index("s")

        @pl.when(sid == 0)
        def _():
            pltpu.sync_copy(zeros_hbm, table_sh)

        plsc.subcore_barrier()

        @pl.when(cid == 0)
        def _():
            _scatter_pipe(table_sh, dst2_hbm, msg0_hbm, n_chunks, 0)

        @pl.when(cid == 1)
        def _():
            _scatter_pipe(table_sh, dst2_hbm, msg1_hbm, n_chunks, 0)

        plsc.subcore_barrier()

        @pl.when((sid == 0) & (cid == 0))
        def _():
            pltpu.sync_copy(table_sh, o0_hbm)

        @pl.when((sid == 0) & (cid == 1))
        def _():
            pltpu.sync_copy(table_sh, o1_hbm)

    return scatter_k


def _make_den_scatter(n_nodes, e_pad):
    half_chunks = e_pad // (2 * G)
    mesh = plsc.VectorSubcoreMesh(core_axis_name="c", subcore_axis_name="s")

    @functools.partial(
        pl.kernel,
        mesh=mesh,
        out_type=[jax.ShapeDtypeStruct((n_nodes, AW), jnp.float32),
                  jax.ShapeDtypeStruct((n_nodes, AW), jnp.float32)],
        scratch_types=[pltpu.VMEM_SHARED((n_nodes, AW), jnp.float32)],
    )
    def den_k(den_hbm, dst2_hbm, zeros_hbm, o0_hbm, o1_hbm, table_sh):
        cid = lax.axis_index("c")
        sid = lax.axis_index("s")

        @pl.when(sid == 0)
        def _():
            pltpu.sync_copy(zeros_hbm, table_sh)

        plsc.subcore_barrier()

        @pl.when(cid == 0)
        def _():
            _scatter_pipe(table_sh, dst2_hbm, den_hbm, half_chunks, 0)

        @pl.when(cid == 1)
        def _():
            _scatter_pipe(table_sh, dst2_hbm, den_hbm, half_chunks, half_chunks)

        plsc.subcore_barrier()

        @pl.when((sid == 0) & (cid == 0))
        def _():
            pltpu.sync_copy(table_sh, o0_hbm)

        @pl.when((sid == 0) & (cid == 1))
        def _():
            pltpu.sync_copy(table_sh, o1_hbm)

    return den_k


def _p6_body(acc0_ref, acc1_ref, den0_ref, den1_ref, x_ref, bias_ref,
             lnw_ref, lnb_ref, o_ref):
    a0 = acc0_ref[...]
    a1 = acc1_ref[...]
    bn = a0.shape[0]
    d8 = den0_ref[:, 0:8] + den1_ref[:, 0:8]
    num = jnp.concatenate([a0, a1], axis=1)
    den = jnp.concatenate(
        [jnp.broadcast_to(d8[:, h:h + 1], (bn, C)) for h in range(H)], axis=1)
    out = num / (den + 1e-30) + bias_ref[...]
    mean_gate = d8[:, 4:5] / jnp.maximum(d8[:, 5:6], 1.0)
    out = out * mean_gate
    mu = jnp.mean(out, axis=-1, keepdims=True)
    var = jnp.mean((out - mu) ** 2, axis=-1, keepdims=True)
    out = (out - mu) * lax.rsqrt(var + 1e-5) * lnw_ref[...] + lnb_ref[...]
    out = out * jax.nn.sigmoid(out)
    o_ref[...] = out + x_ref[...]


def kernel(x, edge_index, edge_attr, W_l, b_l, W_r, b_r, W_e, att, bias,
           eg_W1, eg_b1, eg_W2, eg_b2, ln_w, ln_b):
    n = x.shape[0]
    e = edge_attr.shape[0]
    e_pad = ((e + NTILES * G - 1) // (NTILES * G)) * (NTILES * G)
    pad = e_pad - e
    src_p = jnp.concatenate([edge_index[0].astype(jnp.int32),
                             jnp.zeros((pad,), jnp.int32)])
    dst_p = jnp.concatenate([edge_index[1].astype(jnp.int32),
                             jnp.zeros((pad,), jnp.int32)])
    ea_p = jnp.concatenate([edge_attr, jnp.zeros((pad, ED), edge_attr.dtype)])

    BN = 1000
    xl, xr = pl.pallas_call(
        _p1_body,
        out_shape=[jax.ShapeDtypeStruct((n, HC), jnp.float32)] * 2,
        grid=(n // BN,),
        in_specs=[
            pl.BlockSpec((BN, HC), lambda i: (i, 0)),
            pl.BlockSpec((HC, HC), lambda i: (0, 0)),
            pl.BlockSpec((1, HC), lambda i: (0, 0)),
            pl.BlockSpec((HC, HC), lambda i: (0, 0)),
            pl.BlockSpec((1, HC), lambda i: (0, 0)),
        ],
        out_specs=[pl.BlockSpec((BN, HC), lambda i: (i, 0))] * 2,
    )(x, W_l, b_l.reshape(1, HC), W_r, b_r.reshape(1, HC))

    gather = _make_gather(e_pad)
    xl_src = gather(xl, src_p.reshape(1, e_pad))
    xr_dst = gather(xr, dst_p.reshape(1, e_pad))

    BE = 2048
    n_eblk = e_pad // BE
    idx256 = jnp.arange(HC)
    attbd = jnp.where(idx256[:, None] // C == jnp.arange(8)[None, :],
                      att.reshape(HC)[:, None], 0.0).astype(jnp.bfloat16)
    aux, mglob = pl.pallas_call(
        _p3_body,
        out_shape=[jax.ShapeDtypeStruct((e_pad, 8), jnp.float32),
                   jax.ShapeDtypeStruct((1, 1), jnp.float32)],
        grid=(n_eblk,),
        in_specs=[
            pl.BlockSpec((BE, HC), lambda i: (i, 0)),
            pl.BlockSpec((BE, HC), lambda i: (i, 0)),
            pl.BlockSpec((BE, ED), lambda i: (i, 0)),
            pl.BlockSpec((ED, HC), lambda i: (0, 0)),
            pl.BlockSpec((HC, 8), lambda i: (0, 0)),
            pl.BlockSpec((ED, 2 * ED), lambda i: (0, 0)),
            pl.BlockSpec((1, 2 * ED), lambda i: (0, 0)),
            pl.BlockSpec((1, 2 * ED), lambda i: (0, 0)),
            pl.BlockSpec((1, 1), lambda i: (0, 0)),
        ],
        out_specs=[pl.BlockSpec((BE, 8), lambda i: (i, 0)),
                   pl.BlockSpec((1, 1), lambda i: (0, 0))],
        scratch_shapes=[pltpu.SMEM((1, 1), jnp.float32)],
    )(xl_src, xr_dst, ea_p, W_e.astype(jnp.bfloat16), attbd,
      eg_W1.astype(jnp.bfloat16), eg_b1.reshape(1, 2 * ED),
      eg_W2.reshape(1, 2 * ED), eg_b2.reshape(1, 1))

    msg0, msg1, denrows = pl.pallas_call(
        functools.partial(_p4_body, e, BE),
        out_shape=[jax.ShapeDtypeStruct((e_pad, AW), jnp.float32)] * 3,
        grid=(n_eblk,),
        in_specs=[
            pl.BlockSpec((BE, 8), lambda i: (i, 0)),
            pl.BlockSpec((BE, HC), lambda i: (i, 0)),
            pl.BlockSpec((1, 1), lambda i: (0, 0)),
        ],
        out_specs=[pl.BlockSpec((BE, AW), lambda i: (i, 0))] * 3,
    )(aux, xl_src, mglob)

    zeros_tbl = jnp.zeros((n, AW), jnp.float32)
    dst2 = dst_p.reshape(1, e_pad)
    acc0, acc1 = _make_scatter(n, e_pad)(msg0, msg1, dst2, zeros_tbl)
    den0, den1 = _make_den_scatter(n, e_pad)(denrows, dst2, zeros_tbl)

    out = pl.pallas_call(
        _p6_body,
        out_shape=jax.ShapeDtypeStruct((n, HC), jnp.float32),
        grid=(n // BN,),
        in_specs=[
            pl.BlockSpec((BN, AW), lambda i: (i, 0)),
            pl.BlockSpec((BN, AW), lambda i: (i, 0)),
            pl.BlockSpec((BN, AW), lambda i: (i, 0)),
            pl.BlockSpec((BN, AW), lambda i: (i, 0)),
            pl.BlockSpec((BN, HC), lambda i: (i, 0)),
            pl.BlockSpec((1, HC), lambda i: (0, 0)),
            pl.BlockSpec((1, HC), lambda i: (0, 0)),
            pl.BlockSpec((1, HC), lambda i: (0, 0)),
        ],
        out_specs=pl.BlockSpec((BN, HC), lambda i: (i, 0)),
    )(acc0, acc1, den0, den1, x, bias.reshape(1, HC), ln_w.reshape(1, HC),
      ln_b.reshape(1, HC))
    return out


# speedup vs baseline: 1.1460x; 1.1274x over previous
import functools

import jax
import jax.numpy as jnp
from jax import lax
from jax.experimental import pallas as pl
from jax.experimental.pallas import tpu as pltpu
from jax.experimental.pallas import tpu_sc as plsc

H = 4
C = 64
HC = H * C
ED = 16
AW = 128
G = 128
NTILES = 32


def _pack_bf16_pair(xf32):
    u = jax.lax.bitcast_convert_type(xf32, jnp.uint32)
    r = (u + 0x7FFF + ((u >> 16) & 1)) >> 16
    lo, hi = r[:, 0:128], r[:, 128:256]
    return jax.lax.bitcast_convert_type(lo | (hi << 16), jnp.int32)


def _unpack_bf16_pair(w_i32):
    w = jax.lax.bitcast_convert_type(w_i32, jnp.uint32)
    lo = jax.lax.bitcast_convert_type(w << 16, jnp.float32)
    hi = jax.lax.bitcast_convert_type(w & jnp.uint32(0xFFFF0000), jnp.float32)
    return jnp.concatenate([lo, hi], axis=1)


def _p1_body(x_ref, wl_ref, bl_ref, wr_ref, br_ref, xl_ref, xr_ref):
    xb = x_ref[...]
    xl = jnp.dot(xb, wl_ref[...], preferred_element_type=jnp.float32) + bl_ref[...]
    xr = jnp.dot(xb, wr_ref[...], preferred_element_type=jnp.float32) + br_ref[...]
    xl_ref[...] = _pack_bf16_pair(xl)
    xr_ref[...] = _pack_bf16_pair(xr)


def _make_gather(e_pad):
    n_chunks = e_pad // G
    mesh = plsc.VectorSubcoreMesh(core_axis_name="c", subcore_axis_name="s")

    c0_chunks = (n_chunks * 13 // 20) // 16 * 16

    @functools.partial(
        pl.kernel,
        mesh=mesh,
        out_type=jax.ShapeDtypeStruct((e_pad, 128), jnp.int32),
    )
    def gather_k(tbl_hbm, idx_hbm, o_hbm):
        cid = lax.axis_index("c")

        def pipe(n_ch, ch0):
            def body(idx_vmem, o_vmem):
                pltpu.sync_copy(tbl_hbm.at[idx_vmem.at[0]], o_vmem)

            pltpu.emit_pipeline(
                body,
                grid=(n_ch,),
                in_specs=[pl.BlockSpec((1, G), lambda i: (0, i + ch0))],
                out_specs=[pl.BlockSpec((G, 128), lambda i: (i + ch0, 0))],
                core_axis_name="s",
                dimension_semantics=(pltpu.PARALLEL,),
            )(idx_hbm, o_hbm)

        @pl.when(cid == 0)
        def _():
            pipe(c0_chunks, 0)

        @pl.when(cid == 1)
        def _():
            pipe(n_chunks - c0_chunks, c0_chunks)

    return gather_k


def _p3_body(xl_ref, xr_ref, ea_ref, we_ref, attbd_ref, w1_ref, b1_ref,
             w2_ref, b2_ref, aux_ref, m_ref, m_acc):
    i = pl.program_id(0)
    be = ea_ref.shape[0]
    ea = ea_ref[...]
    ee = jnp.dot(ea.astype(jnp.bfloat16), we_ref[...],
                 preferred_element_type=jnp.float32)
    xl = _unpack_bf16_pair(xl_ref[...])
    xr = _unpack_bf16_pair(xr_ref[...])
    v = xl + xr + ee
    v = jnp.where(v >= 0, v, 0.2 * v)
    alpha8 = jnp.dot(v.astype(jnp.bfloat16), attbd_ref[...],
                     preferred_element_type=jnp.float32)
    alpha = alpha8[:, 0:4]
    g1 = jnp.dot(ea.astype(jnp.bfloat16), w1_ref[...],
                 preferred_element_type=jnp.float32) + b1_ref[...]
    g1 = g1 * jax.nn.sigmoid(g1)
    g2 = jnp.sum(g1 * w2_ref[...], axis=1, keepdims=True) + b2_ref[...]
    gate = jax.nn.sigmoid(g2)
    one = jnp.ones_like(gate)
    zero = jnp.zeros_like(gate)
    aux_ref[...] = jnp.concatenate([alpha, gate, one, zero, zero], axis=1)
    blkmax = jnp.max(alpha)

    @pl.when(i == 0)
    def _():
        m_acc[0, 0] = blkmax

    @pl.when(i > 0)
    def _():
        m_acc[0, 0] = jnp.maximum(m_acc[0, 0], blkmax)

    m_ref[...] = jnp.full((1, 1), m_acc[0, 0], jnp.float32)


def _p4_body(e_real, be, aux_ref, xl_ref, m_ref, msg0_ref, msg1_ref, den_ref):
    i = pl.program_id(0)
    mglob = m_ref[...]
    aux = aux_ref[...]
    xl = _unpack_bf16_pair(xl_ref[...])
    rows = i * be + lax.broadcasted_iota(jnp.int32, (be, 1), 0)
    valid = (rows < e_real).astype(jnp.float32)
    a = jnp.exp(aux[:, 0:4] - mglob) * valid
    gate = aux[:, 4:5] * valid
    one = aux[:, 5:6] * valid
    msg0_ref[...] = jnp.concatenate(
        [xl[:, 0:64] * a[:, 0:1], xl[:, 64:128] * a[:, 1:2]], axis=1)
    msg1_ref[...] = jnp.concatenate(
        [xl[:, 128:192] * a[:, 2:3], xl[:, 192:256] * a[:, 3:4]], axis=1)
    den_ref[...] = jnp.concatenate(
        [a, gate, one, jnp.zeros((be, AW - 6), jnp.float32)], axis=1)


def _scatter_pipe(table_sh, dst2_hbm, msg_hbm, n_chunks, chunk0):
    def body(idx_vmem, msg_vmem):
        pltpu.sync_copy(msg_vmem, table_sh.at[idx_vmem.at[0]], add=True)

    pltpu.emit_pipeline(
        body,
        grid=(n_chunks,),
        in_specs=[pl.BlockSpec((1, G), lambda i: (0, i + chunk0)),
                  pl.BlockSpec((G, AW), lambda i: (i + chunk0, 0))],
        core_axis_name="s",
        dimension_semantics=(pltpu.PARALLEL,),
    )(dst2_hbm, msg_hbm)


def _make_scatter(n_nodes, e_pad):
    n_chunks = e_pad // G
    mesh = plsc.VectorSubcoreMesh(core_axis_name="c", subcore_axis_name="s")

    @functools.partial(
        pl.kernel,
        mesh=mesh,
        out_type=[jax.ShapeDtypeStruct((n_nodes, AW), jnp.float32),
                  jax.ShapeDtypeStruct((n_nodes, AW), jnp.float32)],
        scratch_types=[pltpu.VMEM_SHARED((n_nodes, AW), jnp.float32)],
    )
    def scatter_k(msg0_hbm, msg1_hbm, dst2_hbm, zeros_hbm, o0_hbm, o1_hbm,
                  table_sh):
        cid = lax.axis_index("c")
        sid = lax.axis_index("s")

        @pl.when(sid == 0)
        def _():
            pltpu.sync_copy(zeros_hbm, table_sh)

        plsc.subcore_barrier()

        @pl.when(cid == 0)
        def _():
            _scatter_pipe(table_sh, dst2_hbm, msg0_hbm, n_chunks, 0)

        @pl.when(cid == 1)
        def _():
            _scatter_pipe(table_sh, dst2_hbm, msg1_hbm, n_chunks, 0)

        plsc.subcore_barrier()

        @pl.when((sid == 0) & (cid == 0))
        def _():
            pltpu.sync_copy(table_sh, o0_hbm)

        @pl.when((sid == 0) & (cid == 1))
        def _():
            pltpu.sync_copy(table_sh, o1_hbm)

    return scatter_k


def _make_den_scatter(n_nodes, e_pad):
    half_chunks = e_pad // (2 * G)
    mesh = plsc.VectorSubcoreMesh(core_axis_name="c", subcore_axis_name="s")

    @functools.partial(
        pl.kernel,
        mesh=mesh,
        out_type=[jax.ShapeDtypeStruct((n_nodes, AW), jnp.float32),
                  jax.ShapeDtypeStruct((n_nodes, AW), jnp.float32)],
        scratch_types=[pltpu.VMEM_SHARED((n_nodes, AW), jnp.float32)],
    )
    def den_k(den_hbm, dst2_hbm, zeros_hbm, o0_hbm, o1_hbm, table_sh):
        cid = lax.axis_index("c")
        sid = lax.axis_index("s")

        @pl.when(sid == 0)
        def _():
            pltpu.sync_copy(zeros_hbm, table_sh)

        plsc.subcore_barrier()

        @pl.when(cid == 0)
        def _():
            _scatter_pipe(table_sh, dst2_hbm, den_hbm, half_chunks, 0)

        @pl.when(cid == 1)
        def _():
            _scatter_pipe(table_sh, dst2_hbm, den_hbm, half_chunks, half_chunks)

        plsc.subcore_barrier()

        @pl.when((sid == 0) & (cid == 0))
        def _():
            pltpu.sync_copy(table_sh, o0_hbm)

        @pl.when((sid == 0) & (cid == 1))
        def _():
            pltpu.sync_copy(table_sh, o1_hbm)

    return den_k


def _p6_body(acc0_ref, acc1_ref, den0_ref, den1_ref, x_ref, bias_ref,
             lnw_ref, lnb_ref, o_ref):
    a0 = acc0_ref[...]
    a1 = acc1_ref[...]
    bn = a0.shape[0]
    d8 = den0_ref[:, 0:8] + den1_ref[:, 0:8]
    num = jnp.concatenate([a0, a1], axis=1)
    den = jnp.concatenate(
        [jnp.broadcast_to(d8[:, h:h + 1], (bn, C)) for h in range(H)], axis=1)
    out = num / (den + 1e-30) + bias_ref[...]
    mean_gate = d8[:, 4:5] / jnp.maximum(d8[:, 5:6], 1.0)
    out = out * mean_gate
    mu = jnp.mean(out, axis=-1, keepdims=True)
    var = jnp.mean((out - mu) ** 2, axis=-1, keepdims=True)
    out = (out - mu) * lax.rsqrt(var + 1e-5) * lnw_ref[...] + lnb_ref[...]
    out = out * jax.nn.sigmoid(out)
    o_ref[...] = out + x_ref[...]


def kernel(x, edge_index, edge_attr, W_l, b_l, W_r, b_r, W_e, att, bias,
           eg_W1, eg_b1, eg_W2, eg_b2, ln_w, ln_b):
    n = x.shape[0]
    e = edge_attr.shape[0]
    e_pad = ((e + NTILES * G - 1) // (NTILES * G)) * (NTILES * G)
    pad = e_pad - e
    src_p = jnp.concatenate([edge_index[0].astype(jnp.int32),
                             jnp.zeros((pad,), jnp.int32)])
    dst_p = jnp.concatenate([edge_index[1].astype(jnp.int32),
                             jnp.zeros((pad,), jnp.int32)])
    ea_p = jnp.concatenate([edge_attr, jnp.zeros((pad, ED), edge_attr.dtype)])

    BN = 1000
    xl, xr = pl.pallas_call(
        _p1_body,
        out_shape=[jax.ShapeDtypeStruct((n, 128), jnp.int32)] * 2,
        grid=(n // BN,),
        in_specs=[
            pl.BlockSpec((BN, HC), lambda i: (i, 0)),
            pl.BlockSpec((HC, HC), lambda i: (0, 0)),
            pl.BlockSpec((1, HC), lambda i: (0, 0)),
            pl.BlockSpec((HC, HC), lambda i: (0, 0)),
            pl.BlockSpec((1, HC), lambda i: (0, 0)),
        ],
        out_specs=[pl.BlockSpec((BN, 128), lambda i: (i, 0))] * 2,
    )(x, W_l, b_l.reshape(1, HC), W_r, b_r.reshape(1, HC))

    gather = _make_gather(e_pad)
    xl_src = gather(xl, src_p.reshape(1, e_pad))
    xr_dst = gather(xr, dst_p.reshape(1, e_pad))

    BE = 2048
    n_eblk = e_pad // BE
    idx256 = jnp.arange(HC)
    attbd = jnp.where(idx256[:, None] // C == jnp.arange(8)[None, :],
                      att.reshape(HC)[:, None], 0.0).astype(jnp.bfloat16)
    aux, mglob = pl.pallas_call(
        _p3_body,
        out_shape=[jax.ShapeDtypeStruct((e_pad, 8), jnp.float32),
                   jax.ShapeDtypeStruct((1, 1), jnp.float32)],
        grid=(n_eblk,),
        in_specs=[
            pl.BlockSpec((BE, 128), lambda i: (i, 0)),
            pl.BlockSpec((BE, 128), lambda i: (i, 0)),
            pl.BlockSpec((BE, ED), lambda i: (i, 0)),
            pl.BlockSpec((ED, HC), lambda i: (0, 0)),
            pl.BlockSpec((HC, 8), lambda i: (0, 0)),
            pl.BlockSpec((ED, 2 * ED), lambda i: (0, 0)),
            pl.BlockSpec((1, 2 * ED), lambda i: (0, 0)),
            pl.BlockSpec((1, 2 * ED), lambda i: (0, 0)),
            pl.BlockSpec((1, 1), lambda i: (0, 0)),
        ],
        out_specs=[pl.BlockSpec((BE, 8), lambda i: (i, 0)),
                   pl.BlockSpec((1, 1), lambda i: (0, 0))],
        scratch_shapes=[pltpu.SMEM((1, 1), jnp.float32)],
    )(xl_src, xr_dst, ea_p, W_e.astype(jnp.bfloat16), attbd,
      eg_W1.astype(jnp.bfloat16), eg_b1.reshape(1, 2 * ED),
      eg_W2.reshape(1, 2 * ED), eg_b2.reshape(1, 1))

    msg0, msg1, denrows = pl.pallas_call(
        functools.partial(_p4_body, e, BE),
        out_shape=[jax.ShapeDtypeStruct((e_pad, AW), jnp.float32)] * 3,
        grid=(n_eblk,),
        in_specs=[
            pl.BlockSpec((BE, 8), lambda i: (i, 0)),
            pl.BlockSpec((BE, 128), lambda i: (i, 0)),
            pl.BlockSpec((1, 1), lambda i: (0, 0)),
        ],
        out_specs=[pl.BlockSpec((BE, AW), lambda i: (i, 0))] * 3,
    )(aux, xl_src, mglob)

    zeros_tbl = jnp.zeros((n, AW), jnp.float32)
    dst2 = dst_p.reshape(1, e_pad)
    acc0, acc1 = _make_scatter(n, e_pad)(msg0, msg1, dst2, zeros_tbl)
    den0, den1 = _make_den_scatter(n, e_pad)(denrows, dst2, zeros_tbl)

    out = pl.pallas_call(
        _p6_body,
        out_shape=jax.ShapeDtypeStruct((n, HC), jnp.float32),
        grid=(n // BN,),
        in_specs=[
            pl.BlockSpec((BN, AW), lambda i: (i, 0)),
            pl.BlockSpec((BN, AW), lambda i: (i, 0)),
            pl.BlockSpec((BN, AW), lambda i: (i, 0)),
            pl.BlockSpec((BN, AW), lambda i: (i, 0)),
            pl.BlockSpec((BN, HC), lambda i: (i, 0)),
            pl.BlockSpec((1, HC), lambda i: (0, 0)),
            pl.BlockSpec((1, HC), lambda i: (0, 0)),
            pl.BlockSpec((1, HC), lambda i: (0, 0)),
        ],
        out_specs=pl.BlockSpec((BN, HC), lambda i: (i, 0)),
    )(acc0, acc1, den0, den1, x, bias.reshape(1, HC), ln_w.reshape(1, HC),
      ln_b.reshape(1, HC))
    return out


# speedup vs baseline: 1.2433x; 1.0849x over previous
import functools

import jax
import jax.numpy as jnp
from jax import lax
from jax.experimental import pallas as pl
from jax.experimental.pallas import tpu as pltpu
from jax.experimental.pallas import tpu_sc as plsc

H = 4
C = 64
HC = H * C
ED = 16
AW = 128
G = 128
NTILES = 32
K_SLAB = 4


def _pack_bf16_pair(xf32):
    u = jax.lax.bitcast_convert_type(xf32, jnp.uint32)
    r = (u + 0x7FFF + ((u >> 16) & 1)) >> 16
    lo, hi = r[:, 0:128], r[:, 128:256]
    return jax.lax.bitcast_convert_type(lo | (hi << 16), jnp.int32)


def _unpack_bf16_pair(w_i32):
    w = jax.lax.bitcast_convert_type(w_i32, jnp.uint32)
    lo = jax.lax.bitcast_convert_type(w << 16, jnp.float32)
    hi = jax.lax.bitcast_convert_type(w & jnp.uint32(0xFFFF0000), jnp.float32)
    return jnp.concatenate([lo, hi], axis=1)


def _p1_body(x_ref, wl_ref, bl_ref, wr_ref, br_ref, xl_ref, xr_ref):
    xb = x_ref[...]
    xl = jnp.dot(xb, wl_ref[...], preferred_element_type=jnp.float32) + bl_ref[...]
    xr = jnp.dot(xb, wr_ref[...], preferred_element_type=jnp.float32) + br_ref[...]
    xl_ref[...] = _pack_bf16_pair(xl)
    xr_ref[...] = _pack_bf16_pair(xr)


def _make_gather(n_rows):
    n_chunks = n_rows // G
    c0_chunks = (n_chunks * 3 // 4) // 16 * 16
    mesh = plsc.VectorSubcoreMesh(core_axis_name="c", subcore_axis_name="s")

    @functools.partial(
        pl.kernel,
        mesh=mesh,
        out_type=jax.ShapeDtypeStruct((n_rows, 128), jnp.int32),
    )
    def gather_k(tbl_hbm, idx_hbm, o_hbm):
        cid = lax.axis_index("c")

        def pipe(n_ch, ch0):
            def body(idx_vmem, o_vmem):
                pltpu.sync_copy(tbl_hbm.at[idx_vmem.at[0]], o_vmem)

            pltpu.emit_pipeline(
                body,
                grid=(n_ch,),
                in_specs=[pl.BlockSpec((1, G), lambda i: (0, i + ch0))],
                out_specs=[pl.BlockSpec((G, 128), lambda i: (i + ch0, 0))],
                core_axis_name="s",
                dimension_semantics=(pltpu.PARALLEL,),
            )(idx_hbm, o_hbm)

        @pl.when(cid == 0)
        def _():
            pipe(c0_chunks, 0)

        @pl.when(cid == 1)
        def _():
            pipe(n_chunks - c0_chunks, c0_chunks)

    return gather_k


def _p3_body(xl_ref, xr_ref, ea_ref, we_ref, attbd_ref, w1_ref, b1_ref,
             w2_ref, b2_ref, aux_ref, m_ref, m_acc):
    i = pl.program_id(0)
    ea = ea_ref[...]
    ee = jnp.dot(ea.astype(jnp.bfloat16), we_ref[...],
                 preferred_element_type=jnp.float32)
    xl = _unpack_bf16_pair(xl_ref[...])
    xr = _unpack_bf16_pair(xr_ref[...])
    v = xl + xr + ee
    v = jnp.where(v >= 0, v, 0.2 * v)
    alpha8 = jnp.dot(v.astype(jnp.bfloat16), attbd_ref[...],
                     preferred_element_type=jnp.float32)
    alpha = alpha8[:, 0:4]
    g1 = jnp.dot(ea.astype(jnp.bfloat16), w1_ref[...],
                 preferred_element_type=jnp.float32) + b1_ref[...]
    g1 = g1 * jax.nn.sigmoid(g1)
    g2 = jnp.sum(g1 * w2_ref[...], axis=1, keepdims=True) + b2_ref[...]
    gate = jax.nn.sigmoid(g2)
    one = jnp.ones_like(gate)
    zero = jnp.zeros_like(gate)
    aux_ref[...] = jnp.concatenate([alpha, gate, one, zero, zero], axis=1)
    blkmax = jnp.max(alpha)

    @pl.when(i == 0)
    def _():
        m_acc[0, 0] = blkmax

    @pl.when(i > 0)
    def _():
        m_acc[0, 0] = jnp.maximum(m_acc[0, 0], blkmax)

    m_ref[...] = jnp.full((1, 1), m_acc[0, 0], jnp.float32)


def _p4_body(e_real, base, be, aux_ref, xl_ref, m_ref, msg0_ref, msg1_ref,
             den_ref):
    i = pl.program_id(0)
    mglob = jnp.max(m_ref[...])
    aux = aux_ref[...]
    xl = _unpack_bf16_pair(xl_ref[...])
    rows = base + i * be + lax.broadcasted_iota(jnp.int32, (be, 1), 0)
    valid = (rows < e_real).astype(jnp.float32)
    a = jnp.exp(aux[:, 0:4] - mglob) * valid
    gate = aux[:, 4:5] * valid
    one = aux[:, 5:6] * valid
    msg0_ref[...] = jnp.concatenate(
        [xl[:, 0:64] * a[:, 0:1], xl[:, 64:128] * a[:, 1:2]], axis=1)
    msg1_ref[...] = jnp.concatenate(
        [xl[:, 128:192] * a[:, 2:3], xl[:, 192:256] * a[:, 3:4]], axis=1)
    den_ref[...] = jnp.concatenate(
        [a, gate, one, jnp.zeros((be, AW - 6), jnp.float32)], axis=1)


def _scatter_pipe(table_sh, dst2_hbm, msg_hbm, n_chunks, chunk0):
    def body(idx_vmem, msg_vmem):
        pltpu.sync_copy(msg_vmem, table_sh.at[idx_vmem.at[0]], add=True)

    pltpu.emit_pipeline(
        body,
        grid=(n_chunks,),
        in_specs=[pl.BlockSpec((1, G), lambda i: (0, i + chunk0)),
                  pl.BlockSpec((G, AW), lambda i: (i + chunk0, 0))],
        core_axis_name="s",
        dimension_semantics=(pltpu.PARALLEL,),
    )(dst2_hbm, msg_hbm)


def _make_scatter(n_nodes, slab):
    n_chunks = slab // G
    mesh = plsc.VectorSubcoreMesh(core_axis_name="c", subcore_axis_name="s")

    @functools.partial(
        pl.kernel,
        mesh=mesh,
        out_type=[jax.ShapeDtypeStruct((n_nodes, AW), jnp.float32),
                  jax.ShapeDtypeStruct((n_nodes, AW), jnp.float32)],
        scratch_types=[pltpu.VMEM_SHARED((n_nodes, AW), jnp.float32)],
    )
    def scatter_k(*refs):
        msg0s = refs[0:K_SLAB]
        msg1s = refs[K_SLAB:2 * K_SLAB]
        dsts = refs[2 * K_SLAB:3 * K_SLAB]
        zeros_hbm = refs[3 * K_SLAB]
        o0_hbm, o1_hbm, table_sh = refs[3 * K_SLAB + 1:]
        cid = lax.axis_index("c")
        sid = lax.axis_index("s")

        @pl.when(sid == 0)
        def _():
            pltpu.sync_copy(zeros_hbm, table_sh)

        plsc.subcore_barrier()

        @pl.when(cid == 0)
        def _():
            for k in range(K_SLAB):
                _scatter_pipe(table_sh, dsts[k], msg0s[k], n_chunks, 0)

        @pl.when(cid == 1)
        def _():
            for k in range(K_SLAB):
                _scatter_pipe(table_sh, dsts[k], msg1s[k], n_chunks, 0)

        plsc.subcore_barrier()

        @pl.when((sid == 0) & (cid == 0))
        def _():
            pltpu.sync_copy(table_sh, o0_hbm)

        @pl.when((sid == 0) & (cid == 1))
        def _():
            pltpu.sync_copy(table_sh, o1_hbm)

    return scatter_k


def _make_den_scatter(n_nodes, slab):
    half_chunks = slab // (2 * G)
    mesh = plsc.VectorSubcoreMesh(core_axis_name="c", subcore_axis_name="s")

    @functools.partial(
        pl.kernel,
        mesh=mesh,
        out_type=[jax.ShapeDtypeStruct((n_nodes, AW), jnp.float32),
                  jax.ShapeDtypeStruct((n_nodes, AW), jnp.float32)],
        scratch_types=[pltpu.VMEM_SHARED((n_nodes, AW), jnp.float32)],
    )
    def den_k(*refs):
        dens = refs[0:K_SLAB]
        dsts = refs[K_SLAB:2 * K_SLAB]
        zeros_hbm = refs[2 * K_SLAB]
        o0_hbm, o1_hbm, table_sh = refs[2 * K_SLAB + 1:]
        cid = lax.axis_index("c")
        sid = lax.axis_index("s")

        @pl.when(sid == 0)
        def _():
            pltpu.sync_copy(zeros_hbm, table_sh)

        plsc.subcore_barrier()

        @pl.when(cid == 0)
        def _():
            for k in range(K_SLAB):
                _scatter_pipe(table_sh, dsts[k], dens[k], half_chunks, 0)

        @pl.when(cid == 1)
        def _():
            for k in range(K_SLAB):
                _scatter_pipe(table_sh, dsts[k], dens[k], half_chunks,
                              half_chunks)

        plsc.subcore_barrier()

        @pl.when((sid == 0) & (cid == 0))
        def _():
            pltpu.sync_copy(table_sh, o0_hbm)

        @pl.when((sid == 0) & (cid == 1))
        def _():
            pltpu.sync_copy(table_sh, o1_hbm)

    return den_k


def _p6_body(acc0_ref, acc1_ref, den0_ref, den1_ref, x_ref, bias_ref,
             lnw_ref, lnb_ref, o_ref):
    a0 = acc0_ref[...]
    a1 = acc1_ref[...]
    bn = a0.shape[0]
    d8 = den0_ref[:, 0:8] + den1_ref[:, 0:8]
    num = jnp.concatenate([a0, a1], axis=1)
    den = jnp.concatenate(
        [jnp.broadcast_to(d8[:, h:h + 1], (bn, C)) for h in range(H)], axis=1)
    out = num / (den + 1e-30) + bias_ref[...]
    mean_gate = d8[:, 4:5] / jnp.maximum(d8[:, 5:6], 1.0)
    out = out * mean_gate
    mu = jnp.mean(out, axis=-1, keepdims=True)
    var = jnp.mean((out - mu) ** 2, axis=-1, keepdims=True)
    out = (out - mu) * lax.rsqrt(var + 1e-5) * lnw_ref[...] + lnb_ref[...]
    out = out * jax.nn.sigmoid(out)
    o_ref[...] = out + x_ref[...]


def kernel(x, edge_index, edge_attr, W_l, b_l, W_r, b_r, W_e, att, bias,
           eg_W1, eg_b1, eg_W2, eg_b2, ln_w, ln_b):
    n = x.shape[0]
    e = edge_attr.shape[0]
    quantum = NTILES * G * K_SLAB
    e_pad = ((e + quantum - 1) // quantum) * quantum
    slab = e_pad // K_SLAB
    pad = e_pad - e
    src_p = jnp.concatenate([edge_index[0].astype(jnp.int32),
                             jnp.zeros((pad,), jnp.int32)])
    dst_p = jnp.concatenate([edge_index[1].astype(jnp.int32),
                             jnp.zeros((pad,), jnp.int32)])
    ea_p = jnp.concatenate([edge_attr, jnp.zeros((pad, ED), edge_attr.dtype)])
    src_s = [src_p[k * slab:(k + 1) * slab].reshape(1, slab)
             for k in range(K_SLAB)]
    dst_s = [dst_p[k * slab:(k + 1) * slab].reshape(1, slab)
             for k in range(K_SLAB)]
    ea_s = [ea_p[k * slab:(k + 1) * slab] for k in range(K_SLAB)]

    BN = 1000
    xl, xr = pl.pallas_call(
        _p1_body,
        out_shape=[jax.ShapeDtypeStruct((n, 128), jnp.int32)] * 2,
        grid=(n // BN,),
        in_specs=[
            pl.BlockSpec((BN, HC), lambda i: (i, 0)),
            pl.BlockSpec((HC, HC), lambda i: (0, 0)),
            pl.BlockSpec((1, HC), lambda i: (0, 0)),
            pl.BlockSpec((HC, HC), lambda i: (0, 0)),
            pl.BlockSpec((1, HC), lambda i: (0, 0)),
        ],
        out_specs=[pl.BlockSpec((BN, 128), lambda i: (i, 0))] * 2,
    )(x, W_l, b_l.reshape(1, HC), W_r, b_r.reshape(1, HC))

    gather = _make_gather(slab)
    BE = 2048
    n_eblk = slab // BE
    idx256 = jnp.arange(HC)
    attbd = jnp.where(idx256[:, None] // C == jnp.arange(8)[None, :],
                      att.reshape(HC)[:, None], 0.0).astype(jnp.bfloat16)

    p3 = pl.pallas_call(
        _p3_body,
        out_shape=[jax.ShapeDtypeStruct((slab, 8), jnp.float32),
                   jax.ShapeDtypeStruct((1, 1), jnp.float32)],
        grid=(n_eblk,),
        in_specs=[
            pl.BlockSpec((BE, 128), lambda i: (i, 0)),
            pl.BlockSpec((BE, 128), lambda i: (i, 0)),
            pl.BlockSpec((BE, ED), lambda i: (i, 0)),
            pl.BlockSpec((ED, HC), lambda i: (0, 0)),
            pl.BlockSpec((HC, 8), lambda i: (0, 0)),
            pl.BlockSpec((ED, 2 * ED), lambda i: (0, 0)),
            pl.BlockSpec((1, 2 * ED), lambda i: (0, 0)),
            pl.BlockSpec((1, 2 * ED), lambda i: (0, 0)),
            pl.BlockSpec((1, 1), lambda i: (0, 0)),
        ],
        out_specs=[pl.BlockSpec((BE, 8), lambda i: (i, 0)),
                   pl.BlockSpec((1, 1), lambda i: (0, 0))],
        scratch_shapes=[pltpu.SMEM((1, 1), jnp.float32)],
    )

    xl_s, xr_s, aux_s, max_s = [], [], [], []
    for kk in range(K_SLAB):
        xls = gather(xl, src_s[kk])
        xrs = gather(xr, dst_s[kk])
        aux_k, m_k = p3(xls, xrs, ea_s[kk], W_e.astype(jnp.bfloat16), attbd,
                        eg_W1.astype(jnp.bfloat16), eg_b1.reshape(1, 2 * ED),
                        eg_W2.reshape(1, 2 * ED), eg_b2.reshape(1, 1))
        xl_s.append(xls)
        xr_s.append(xrs)
        aux_s.append(aux_k)
        max_s.append(m_k)
    maxes = jnp.concatenate(max_s, axis=0)

    msg0_s, msg1_s, den_s = [], [], []
    for kk in range(K_SLAB):
        m0, m1, dn = pl.pallas_call(
            functools.partial(_p4_body, e, kk * slab, BE),
            out_shape=[jax.ShapeDtypeStruct((slab, AW), jnp.float32)] * 3,
            grid=(n_eblk,),
            in_specs=[
                pl.BlockSpec((BE, 8), lambda i: (i, 0)),
                pl.BlockSpec((BE, 128), lambda i: (i, 0)),
                pl.BlockSpec((K_SLAB, 1), lambda i: (0, 0)),
            ],
            out_specs=[pl.BlockSpec((BE, AW), lambda i: (i, 0))] * 3,
        )(aux_s[kk], xl_s[kk], maxes)
        msg0_s.append(m0)
        msg1_s.append(m1)
        den_s.append(dn)

    zeros_tbl = jnp.zeros((n, AW), jnp.float32)
    acc0, acc1 = _make_scatter(n, slab)(
        *msg0_s, *msg1_s, *dst_s, zeros_tbl)
    den0, den1 = _make_den_scatter(n, slab)(*den_s, *dst_s, zeros_tbl)

    out = pl.pallas_call(
        _p6_body,
        out_shape=jax.ShapeDtypeStruct((n, HC), jnp.float32),
        grid=(n // BN,),
        in_specs=[
            pl.BlockSpec((BN, AW), lambda i: (i, 0)),
            pl.BlockSpec((BN, AW), lambda i: (i, 0)),
            pl.BlockSpec((BN, AW), lambda i: (i, 0)),
            pl.BlockSpec((BN, AW), lambda i: (i, 0)),
            pl.BlockSpec((BN, HC), lambda i: (i, 0)),
            pl.BlockSpec((1, HC), lambda i: (0, 0)),
            pl.BlockSpec((1, HC), lambda i: (0, 0)),
            pl.BlockSpec((1, HC), lambda i: (0, 0)),
        ],
        out_specs=pl.BlockSpec((BN, HC), lambda i: (i, 0)),
    )(acc0, acc1, den0, den1, x, bias.reshape(1, HC), ln_w.reshape(1, HC),
      ln_b.reshape(1, HC))
    return out


# speedup vs baseline: 1.8889x; 1.5193x over previous
import functools

import jax
import jax.numpy as jnp
from jax import lax
from jax.experimental import pallas as pl
from jax.experimental.pallas import tpu as pltpu
from jax.experimental.pallas import tpu_sc as plsc

H = 4
C = 64
HC = H * C
ED = 16
AW = 128
G = 128
NTILES = 32
K_SLAB = 4


def _pack_bf16_pair(xf32):
    u = jax.lax.bitcast_convert_type(xf32, jnp.uint32)
    r = (u + 0x7FFF + ((u >> 16) & 1)) >> 16
    lo, hi = r[:, 0:128], r[:, 128:256]
    return jax.lax.bitcast_convert_type(lo | (hi << 16), jnp.int32)


def _unpack_bf16_pair(w_i32):
    w = jax.lax.bitcast_convert_type(w_i32, jnp.uint32)
    lo = jax.lax.bitcast_convert_type(w << 16, jnp.float32)
    hi = jax.lax.bitcast_convert_type(w & jnp.uint32(0xFFFF0000), jnp.float32)
    return jnp.concatenate([lo, hi], axis=1)


def _p1_body(x_ref, wl_ref, bl_ref, wr_ref, br_ref, xl_ref, xr_ref):
    xb = x_ref[...]
    xl = jnp.dot(xb, wl_ref[...], preferred_element_type=jnp.float32) + bl_ref[...]
    xr = jnp.dot(xb, wr_ref[...], preferred_element_type=jnp.float32) + br_ref[...]
    xl_ref[...] = _pack_bf16_pair(xl)
    xr_ref[...] = _pack_bf16_pair(xr)


def _make_gather(n_tbl, n_rows):
    n_chunks = n_rows // G
    mesh = plsc.VectorSubcoreMesh(core_axis_name="c", subcore_axis_name="s")

    @functools.partial(
        pl.kernel,
        mesh=mesh,
        out_type=jax.ShapeDtypeStruct((n_rows, 128), jnp.int32),
        scratch_types=[pltpu.VMEM_SHARED((n_tbl, 128), jnp.int32)],
    )
    def gather_k(tbl_hbm, idx_hbm, o_hbm, table_sh):
        sid = lax.axis_index("s")

        @pl.when(sid == 0)
        def _():
            pltpu.sync_copy(tbl_hbm, table_sh)

        plsc.subcore_barrier()

        def body(idx_vmem, o_vmem):
            pltpu.sync_copy(table_sh.at[idx_vmem.at[0]], o_vmem)

        pltpu.emit_pipeline(
            body,
            grid=(n_chunks,),
            in_specs=[pl.BlockSpec((1, G), lambda i: (0, i))],
            out_specs=[pl.BlockSpec((G, 128), lambda i: (i, 0))],
            core_axis_name=("c", "s"),
            dimension_semantics=(pltpu.PARALLEL,),
        )(idx_hbm, o_hbm)

    return gather_k


def _p3_body(xl_ref, xr_ref, ea_ref, we_ref, attbd_ref, w1_ref, b1_ref,
             w2_ref, b2_ref, aux_ref, m_ref, m_acc):
    i = pl.program_id(0)
    ea = ea_ref[...]
    ee = jnp.dot(ea.astype(jnp.bfloat16), we_ref[...],
                 preferred_element_type=jnp.float32)
    xl = _unpack_bf16_pair(xl_ref[...])
    xr = _unpack_bf16_pair(xr_ref[...])
    v = xl + xr + ee
    v = jnp.where(v >= 0, v, 0.2 * v)
    alpha8 = jnp.dot(v.astype(jnp.bfloat16), attbd_ref[...],
                     preferred_element_type=jnp.float32)
    alpha = alpha8[:, 0:4]
    g1 = jnp.dot(ea.astype(jnp.bfloat16), w1_ref[...],
                 preferred_element_type=jnp.float32) + b1_ref[...]
    g1 = g1 * jax.nn.sigmoid(g1)
    g2 = jnp.sum(g1 * w2_ref[...], axis=1, keepdims=True) + b2_ref[...]
    gate = jax.nn.sigmoid(g2)
    one = jnp.ones_like(gate)
    zero = jnp.zeros_like(gate)
    aux_ref[...] = jnp.concatenate([alpha, gate, one, zero, zero], axis=1)
    blkmax = jnp.max(alpha)

    @pl.when(i == 0)
    def _():
        m_acc[0, 0] = blkmax

    @pl.when(i > 0)
    def _():
        m_acc[0, 0] = jnp.maximum(m_acc[0, 0], blkmax)

    m_ref[...] = jnp.full((1, 1), m_acc[0, 0], jnp.float32)


def _p4_body(e_real, base, be, aux_ref, xl_ref, m_ref, msg0_ref, msg1_ref,
             den_ref):
    i = pl.program_id(0)
    mglob = jnp.max(m_ref[...])
    aux = aux_ref[...]
    xl = _unpack_bf16_pair(xl_ref[...])
    rows = base + i * be + lax.broadcasted_iota(jnp.int32, (be, 1), 0)
    valid = (rows < e_real).astype(jnp.float32)
    a = jnp.exp(aux[:, 0:4] - mglob) * valid
    gate = aux[:, 4:5] * valid
    one = aux[:, 5:6] * valid
    msg0_ref[...] = jnp.concatenate(
        [xl[:, 0:64] * a[:, 0:1], xl[:, 64:128] * a[:, 1:2]], axis=1)
    msg1_ref[...] = jnp.concatenate(
        [xl[:, 128:192] * a[:, 2:3], xl[:, 192:256] * a[:, 3:4]], axis=1)
    den_ref[...] = jnp.concatenate(
        [a, gate, one, jnp.zeros((be, AW - 6), jnp.float32)], axis=1)


def _scatter_pipe(table_sh, dst2_hbm, msg_hbm, n_chunks, chunk0):
    def body(idx_vmem, msg_vmem):
        pltpu.sync_copy(msg_vmem, table_sh.at[idx_vmem.at[0]], add=True)

    pltpu.emit_pipeline(
        body,
        grid=(n_chunks,),
        in_specs=[pl.BlockSpec((1, G), lambda i: (0, i + chunk0)),
                  pl.BlockSpec((G, AW), lambda i: (i + chunk0, 0))],
        core_axis_name="s",
        dimension_semantics=(pltpu.PARALLEL,),
    )(dst2_hbm, msg_hbm)


def _make_scatter(n_nodes, slab):
    n_chunks = slab // G
    mesh = plsc.VectorSubcoreMesh(core_axis_name="c", subcore_axis_name="s")

    @functools.partial(
        pl.kernel,
        mesh=mesh,
        out_type=[jax.ShapeDtypeStruct((n_nodes, AW), jnp.float32),
                  jax.ShapeDtypeStruct((n_nodes, AW), jnp.float32)],
        scratch_types=[pltpu.VMEM_SHARED((n_nodes, AW), jnp.float32)],
    )
    def scatter_k(*refs):
        msg0s = refs[0:K_SLAB]
        msg1s = refs[K_SLAB:2 * K_SLAB]
        dsts = refs[2 * K_SLAB:3 * K_SLAB]
        zeros_hbm = refs[3 * K_SLAB]
        o0_hbm, o1_hbm, table_sh = refs[3 * K_SLAB + 1:]
        cid = lax.axis_index("c")
        sid = lax.axis_index("s")

        @pl.when(sid == 0)
        def _():
            pltpu.sync_copy(zeros_hbm, table_sh)

        plsc.subcore_barrier()

        @pl.when(cid == 0)
        def _():
            for k in range(K_SLAB):
                _scatter_pipe(table_sh, dsts[k], msg0s[k], n_chunks, 0)

        @pl.when(cid == 1)
        def _():
            for k in range(K_SLAB):
                _scatter_pipe(table_sh, dsts[k], msg1s[k], n_chunks, 0)

        plsc.subcore_barrier()

        @pl.when((sid == 0) & (cid == 0))
        def _():
            pltpu.sync_copy(table_sh, o0_hbm)

        @pl.when((sid == 0) & (cid == 1))
        def _():
            pltpu.sync_copy(table_sh, o1_hbm)

    return scatter_k


def _make_den_scatter(n_nodes, slab):
    half_chunks = slab // (2 * G)
    mesh = plsc.VectorSubcoreMesh(core_axis_name="c", subcore_axis_name="s")

    @functools.partial(
        pl.kernel,
        mesh=mesh,
        out_type=[jax.ShapeDtypeStruct((n_nodes, AW), jnp.float32),
                  jax.ShapeDtypeStruct((n_nodes, AW), jnp.float32)],
        scratch_types=[pltpu.VMEM_SHARED((n_nodes, AW), jnp.float32)],
    )
    def den_k(*refs):
        dens = refs[0:K_SLAB]
        dsts = refs[K_SLAB:2 * K_SLAB]
        zeros_hbm = refs[2 * K_SLAB]
        o0_hbm, o1_hbm, table_sh = refs[2 * K_SLAB + 1:]
        cid = lax.axis_index("c")
        sid = lax.axis_index("s")

        @pl.when(sid == 0)
        def _():
            pltpu.sync_copy(zeros_hbm, table_sh)

        plsc.subcore_barrier()

        @pl.when(cid == 0)
        def _():
            for k in range(K_SLAB):
                _scatter_pipe(table_sh, dsts[k], dens[k], half_chunks, 0)

        @pl.when(cid == 1)
        def _():
            for k in range(K_SLAB):
                _scatter_pipe(table_sh, dsts[k], dens[k], half_chunks,
                              half_chunks)

        plsc.subcore_barrier()

        @pl.when((sid == 0) & (cid == 0))
        def _():
            pltpu.sync_copy(table_sh, o0_hbm)

        @pl.when((sid == 0) & (cid == 1))
        def _():
            pltpu.sync_copy(table_sh, o1_hbm)

    return den_k


def _p6_body(acc0_ref, acc1_ref, den0_ref, den1_ref, x_ref, bias_ref,
             lnw_ref, lnb_ref, o_ref):
    a0 = acc0_ref[...]
    a1 = acc1_ref[...]
    bn = a0.shape[0]
    d8 = den0_ref[:, 0:8] + den1_ref[:, 0:8]
    num = jnp.concatenate([a0, a1], axis=1)
    den = jnp.concatenate(
        [jnp.broadcast_to(d8[:, h:h + 1], (bn, C)) for h in range(H)], axis=1)
    out = num / (den + 1e-30) + bias_ref[...]
    mean_gate = d8[:, 4:5] / jnp.maximum(d8[:, 5:6], 1.0)
    out = out * mean_gate
    mu = jnp.mean(out, axis=-1, keepdims=True)
    var = jnp.mean((out - mu) ** 2, axis=-1, keepdims=True)
    out = (out - mu) * lax.rsqrt(var + 1e-5) * lnw_ref[...] + lnb_ref[...]
    out = out * jax.nn.sigmoid(out)
    o_ref[...] = out + x_ref[...]


def kernel(x, edge_index, edge_attr, W_l, b_l, W_r, b_r, W_e, att, bias,
           eg_W1, eg_b1, eg_W2, eg_b2, ln_w, ln_b):
    n = x.shape[0]
    e = edge_attr.shape[0]
    quantum = NTILES * G * K_SLAB
    e_pad = ((e + quantum - 1) // quantum) * quantum
    slab = e_pad // K_SLAB
    pad = e_pad - e
    src_p = jnp.concatenate([edge_index[0].astype(jnp.int32),
                             jnp.zeros((pad,), jnp.int32)])
    dst_p = jnp.concatenate([edge_index[1].astype(jnp.int32),
                             jnp.zeros((pad,), jnp.int32)])
    ea_p = jnp.concatenate([edge_attr, jnp.zeros((pad, ED), edge_attr.dtype)])
    src_s = [src_p[k * slab:(k + 1) * slab].reshape(1, slab)
             for k in range(K_SLAB)]
    dst_s = [dst_p[k * slab:(k + 1) * slab].reshape(1, slab)
             for k in range(K_SLAB)]
    ea_s = [ea_p[k * slab:(k + 1) * slab] for k in range(K_SLAB)]

    BN = 1000
    xl, xr = pl.pallas_call(
        _p1_body,
        out_shape=[jax.ShapeDtypeStruct((n, 128), jnp.int32)] * 2,
        grid=(n // BN,),
        in_specs=[
            pl.BlockSpec((BN, HC), lambda i: (i, 0)),
            pl.BlockSpec((HC, HC), lambda i: (0, 0)),
            pl.BlockSpec((1, HC), lambda i: (0, 0)),
            pl.BlockSpec((HC, HC), lambda i: (0, 0)),
            pl.BlockSpec((1, HC), lambda i: (0, 0)),
        ],
        out_specs=[pl.BlockSpec((BN, 128), lambda i: (i, 0))] * 2,
    )(x, W_l, b_l.reshape(1, HC), W_r, b_r.reshape(1, HC))

    gather = _make_gather(n, slab)
    BE = 2048
    n_eblk = slab // BE
    idx256 = jnp.arange(HC)
    attbd = jnp.where(idx256[:, None] // C == jnp.arange(8)[None, :],
                      att.reshape(HC)[:, None], 0.0).astype(jnp.bfloat16)

    p3 = pl.pallas_call(
        _p3_body,
        out_shape=[jax.ShapeDtypeStruct((slab, 8), jnp.float32),
                   jax.ShapeDtypeStruct((1, 1), jnp.float32)],
        grid=(n_eblk,),
        in_specs=[
            pl.BlockSpec((BE, 128), lambda i: (i, 0)),
            pl.BlockSpec((BE, 128), lambda i: (i, 0)),
            pl.BlockSpec((BE, ED), lambda i: (i, 0)),
            pl.BlockSpec((ED, HC), lambda i: (0, 0)),
            pl.BlockSpec((HC, 8), lambda i: (0, 0)),
            pl.BlockSpec((ED, 2 * ED), lambda i: (0, 0)),
            pl.BlockSpec((1, 2 * ED), lambda i: (0, 0)),
            pl.BlockSpec((1, 2 * ED), lambda i: (0, 0)),
            pl.BlockSpec((1, 1), lambda i: (0, 0)),
        ],
        out_specs=[pl.BlockSpec((BE, 8), lambda i: (i, 0)),
                   pl.BlockSpec((1, 1), lambda i: (0, 0))],
        scratch_shapes=[pltpu.SMEM((1, 1), jnp.float32)],
    )

    xl_s, xr_s, aux_s, max_s = [], [], [], []
    for kk in range(K_SLAB):
        xls = gather(xl, src_s[kk])
        xrs = gather(xr, dst_s[kk])
        aux_k, m_k = p3(xls, xrs, ea_s[kk], W_e.astype(jnp.bfloat16), attbd,
                        eg_W1.astype(jnp.bfloat16), eg_b1.reshape(1, 2 * ED),
                        eg_W2.reshape(1, 2 * ED), eg_b2.reshape(1, 1))
        xl_s.append(xls)
        xr_s.append(xrs)
        aux_s.append(aux_k)
        max_s.append(m_k)
    maxes = jnp.concatenate(max_s, axis=0)

    msg0_s, msg1_s, den_s = [], [], []
    for kk in range(K_SLAB):
        m0, m1, dn = pl.pallas_call(
            functools.partial(_p4_body, e, kk * slab, BE),
            out_shape=[jax.ShapeDtypeStruct((slab, AW), jnp.float32)] * 3,
            grid=(n_eblk,),
            in_specs=[
                pl.BlockSpec((BE, 8), lambda i: (i, 0)),
                pl.BlockSpec((BE, 128), lambda i: (i, 0)),
                pl.BlockSpec((K_SLAB, 1), lambda i: (0, 0)),
            ],
            out_specs=[pl.BlockSpec((BE, AW), lambda i: (i, 0))] * 3,
        )(aux_s[kk], xl_s[kk], maxes)
        msg0_s.append(m0)
        msg1_s.append(m1)
        den_s.append(dn)

    zeros_tbl = jnp.zeros((n, AW), jnp.float32)
    acc0, acc1 = _make_scatter(n, slab)(
        *msg0_s, *msg1_s, *dst_s, zeros_tbl)
    den0, den1 = _make_den_scatter(n, slab)(*den_s, *dst_s, zeros_tbl)

    out = pl.pallas_call(
        _p6_body,
        out_shape=jax.ShapeDtypeStruct((n, HC), jnp.float32),
        grid=(n // BN,),
        in_specs=[
            pl.BlockSpec((BN, AW), lambda i: (i, 0)),
            pl.BlockSpec((BN, AW), lambda i: (i, 0)),
            pl.BlockSpec((BN, AW), lambda i: (i, 0)),
            pl.BlockSpec((BN, AW), lambda i: (i, 0)),
            pl.BlockSpec((BN, HC), lambda i: (i, 0)),
            pl.BlockSpec((1, HC), lambda i: (0, 0)),
            pl.BlockSpec((1, HC), lambda i: (0, 0)),
            pl.BlockSpec((1, HC), lambda i: (0, 0)),
        ],
        out_specs=pl.BlockSpec((BN, HC), lambda i: (i, 0)),
    )(acc0, acc1, den0, den1, x, bias.reshape(1, HC), ln_w.reshape(1, HC),
      ln_b.reshape(1, HC))
    return out
